# baseline (speedup 1.0000x reference)
import jax
import jax.numpy as jnp
from jax import lax
from jax.experimental import pallas as pl
from jax.experimental.pallas import tpu as pltpu
from jax.experimental.pallas import tpu_sc as plsc

_D = 64
_BATCH = 1024
_SEQ = 1024
_B = _BATCH * _SEQ
_V = 1000000
_NC = 2
_NS = 16
_NW = _NC * _NS
_BPW = _B // _NW
_IW = 128
_NBLK = _V // _IW
_TAILV = _NBLK * _IW
_ROWS_PW = _BPW // _SEQ


def _fmt_body(tblt, tailp, ids, fmt, idxh, ibuf0, ibuf1, obuf0, obuf1,
              idsblk, idxblk, isem0, isem1, osem0, osem1):
    wid = lax.axis_index("s") * _NC + lax.axis_index("c")

    iota16 = lax.iota(jnp.int32, 16)
    _rot = [(iota16 + s) & 15 for s in range(16)]
    _krot = [lax.shift_right_logical(r, 1) for r in _rot]
    _offrot = [lax.shift_left(r & 1, 6) + iota16 for r in _rot]
    _dvecs = [iota16 + 16 * dg for dg in range(4)]

    def transpose_block(ibuf, obuf, nrows):
        ntg = (2 * nrows) // 16

        def tgstep(tg, carry):
            for dg in range(4):
                rowv = _dvecs[dg]
                for s in range(16):
                    colv = _rot[s] + (16 * tg)
                    vals = plsc.load_gather(ibuf, [rowv, colv])
                    kv = _krot[s] + (8 * tg)
                    offv = _offrot[s] + (16 * dg)
                    plsc.store_scatter(obuf, [kv, offv], vals)
            return carry

        lax.fori_loop(0, ntg, tgstep, 0)

    def sup_body(sup, carry):
        r0 = wid * _ROWS_PW + sup * 8
        pltpu.sync_copy(ids.at[pl.ds(r0, 8)], idsblk)

        def shift16(i, carry2):
            r = i // 64
            c = (i % 64) * 16
            idxblk[r, pl.ds(c, 16)] = lax.shift_right_logical(
                idsblk[r, pl.ds(c, 16)], 1)
            return carry2

        lax.fori_loop(0, 512, shift16, 0)
        pltpu.sync_copy(idxblk, idxh.at[pl.ds(r0, 8)])
        return carry

    lax.fori_loop(0, _ROWS_PW // 8, sup_body, 0)

    def fire_in(c, ibuf, sem):
        pltpu.async_copy(tblt.at[:, pl.ds(c * _IW, _IW)], ibuf, sem)

    def drain_in(ibuf, sem):
        pltpu.make_async_copy(tblt.at[:, pl.ds(0, _IW)], ibuf, sem).wait()

    def fire_out(c, obuf, sem):
        pltpu.async_copy(obuf, fmt.at[pl.ds(c * 64, 64)], sem)

    def drain_out(obuf, sem):
        pltpu.make_async_copy(obuf, fmt.at[pl.ds(0, 64)], sem).wait()

    nblk_w = _NBLK // _NW
    c0 = wid * nblk_w
    fire_in(c0, ibuf0, isem0)

    def pairstep(k, carry):
        c = c0 + 2 * k
        drain_in(ibuf0, isem0)
        fire_in(c + 1, ibuf1, isem1)
        transpose_block(ibuf0, obuf0, 64)
        fire_out(c, obuf0, osem0)
        drain_in(ibuf1, isem1)

        @pl.when(c + 2 < c0 + nblk_w)
        def _():
            fire_in(c + 2, ibuf0, isem0)

        transpose_block(ibuf1, obuf1, 64)
        fire_out(c + 1, obuf1, osem1)
        drain_out(obuf0, osem0)
        drain_out(obuf1, osem1)
        return carry

    lax.fori_loop(0, nblk_w // 2, pairstep, 0)

    @pl.when(wid < _NBLK - nblk_w * _NW)
    def _():
        c = nblk_w * _NW + wid
        pltpu.sync_copy(tblt.at[:, pl.ds(c * _IW, _IW)], ibuf0)
        transpose_block(ibuf0, obuf0, 64)
        pltpu.sync_copy(obuf0, fmt.at[pl.ds(c * 64, 64)])

    @pl.when(wid == _NW - 1)
    def _():
        pltpu.sync_copy(tailp, ibuf0)
        transpose_block(ibuf0, obuf0, 32)
        pltpu.sync_copy(obuf0.at[pl.ds(0, 32)], fmt.at[pl.ds(_TAILV // 2, 32)])


def _gather_body(ids, idxh, fmt, out3, idsblk, idxblk, pairs0, pairs1,
                 obuf0, obuf1, hbuf, gsem0, gsem1, wsem0, wsem1):
    wid = lax.axis_index("s") * _NC + lax.axis_index("c")
    row0 = wid * _ROWS_PW

    iota16 = lax.iota(jnp.int32, 16)
    _rot = [(iota16 + s) & 15 for s in range(16)]
    _krot = [lax.shift_right_logical(r, 1) for r in _rot]
    _offrot = [lax.shift_left(r & 1, 6) + iota16 for r in _rot]
    _dvecs = [iota16 + 16 * dg for dg in range(4)]

    def fire_g(cc, pairs, sem):
        r = cc // 8
        c = (cc % 8) * _IW
        pltpu.async_copy(fmt.at[idxblk.at[r, pl.ds(c, _IW)]], pairs, sem)

    def drain_g(pairs, sem):
        pltpu.make_async_copy(fmt.at[pl.ds(0, _IW)], pairs, sem).wait()

    def select_t(cc, pairs, obuf, hbuf):
        r = cc // 8
        c = (cc % 8) * _IW
        def hstep(tg, carry):
            hbuf[pl.ds(16 * tg, 16)] = lax.shift_left(
                lax.bitwise_and(idsblk[r, pl.ds(c + 16 * tg, 16)], 1), 6)
            return carry

        lax.fori_loop(0, 8, hstep, 0)

        def tgstep(tg, carry):
            for s in range(16):
                rowv = _rot[s] + (16 * tg)
                hperm = plsc.load_gather(hbuf, [rowv])
                for dg in range(4):
                    colv = hperm + _dvecs[dg]
                    vals = plsc.load_gather(pairs, [rowv, colv])
                    plsc.store_scatter(obuf, [_dvecs[dg], rowv], vals)
            return carry

        lax.fori_loop(0, 8, tgstep, 0)

    def fire_w(cc, sup, obuf, sem):
        row = row0 + sup * 8 + cc // 8
        pltpu.async_copy(obuf, out3.at[row, :, pl.ds((cc % 8) * _IW, _IW)], sem)

    def drain_w(obuf, sem):
        pltpu.make_async_copy(obuf, out3.at[0, :, pl.ds(0, _IW)], sem).wait()

    def sup_body(sup, carry):
        r0 = row0 + sup * 8
        pltpu.sync_copy(ids.at[pl.ds(r0, 8)], idsblk)
        pltpu.sync_copy(idxh.at[pl.ds(r0, 8)], idxblk)

        fire_g(0, pairs0, gsem0)

        def pairstep(k, carry2):
            cc = 2 * k
            drain_g(pairs0, gsem0)
            fire_g(cc + 1, pairs1, gsem1)
            select_t(cc, pairs0, obuf0, hbuf)
            fire_w(cc, sup, obuf0, wsem0)
            drain_g(pairs1, gsem1)

            @pl.when(cc + 2 < 64)
            def _():
                fire_g(cc + 2, pairs0, gsem0)

            select_t(cc + 1, pairs1, obuf1, hbuf)
            fire_w(cc + 1, sup, obuf1, wsem1)
            drain_w(obuf0, wsem0)
            drain_w(obuf1, wsem1)
            return carry2

        lax.fori_loop(0, 32, pairstep, 0)
        return carry

    lax.fori_loop(0, _ROWS_PW // 8, sup_body, 0)


@jax.jit
def kernel(input_ids, attention_mask, embedding_weight):
    tblt = embedding_weight.T
    tail = lax.slice(embedding_weight, (_TAILV, 0), (_V, _D))
    tailp = jnp.concatenate([tail.T, tail.T], axis=1)

    mesh = plsc.VectorSubcoreMesh(core_axis_name="c", subcore_axis_name="s")
    fmt, idxh = pl.kernel(
        _fmt_body,
        mesh=mesh,
        out_type=(
            jax.ShapeDtypeStruct((_V // 2, 128), jnp.float32),
            jax.ShapeDtypeStruct((_BATCH, _SEQ), jnp.int32),
        ),
        scratch_types=[
            pltpu.VMEM((_D, _IW), jnp.float32),
            pltpu.VMEM((_D, _IW), jnp.float32),
            pltpu.VMEM((_D, _IW), jnp.float32),
            pltpu.VMEM((_D, _IW), jnp.float32),
            pltpu.VMEM((8, _SEQ), jnp.int32),
            pltpu.VMEM((8, _SEQ), jnp.int32),
            pltpu.SemaphoreType.DMA,
            pltpu.SemaphoreType.DMA,
            pltpu.SemaphoreType.DMA,
            pltpu.SemaphoreType.DMA,
        ],
        compiler_params=pltpu.CompilerParams(use_tc_tiling_on_sc=True, needs_layout_passes=False),
    )(tblt, tailp, input_ids)

    out3 = pl.kernel(
        _gather_body,
        mesh=mesh,
        out_type=jax.ShapeDtypeStruct((_BATCH, _D, _SEQ), jnp.float32),
        scratch_types=[
            pltpu.VMEM((8, _SEQ), jnp.int32),
            pltpu.VMEM((8, _SEQ), jnp.int32),
            pltpu.VMEM((_IW, 128), jnp.float32),
            pltpu.VMEM((_IW, 128), jnp.float32),
            pltpu.VMEM((_D, _IW), jnp.float32),
            pltpu.VMEM((_D, _IW), jnp.float32),
            pltpu.VMEM((_IW,), jnp.int32),
            pltpu.SemaphoreType.DMA,
            pltpu.SemaphoreType.DMA,
            pltpu.SemaphoreType.DMA,
            pltpu.SemaphoreType.DMA,
        ],
        compiler_params=pltpu.CompilerParams(use_tc_tiling_on_sc=True, needs_layout_passes=False),
    )(input_ids, idxh, fmt)

    token_embeddings = out3.transpose(0, 2, 1)
    return (input_ids, token_embeddings, attention_mask)


# speedup vs baseline: 1.0061x; 1.0061x over previous
import jax
import jax.numpy as jnp
from jax import lax
from jax.experimental import pallas as pl
from jax.experimental.pallas import tpu as pltpu
from jax.experimental.pallas import tpu_sc as plsc

_D = 64
_BATCH = 1024
_SEQ = 1024
_B = _BATCH * _SEQ
_V = 1000000
_NC = 2
_NS = 16
_NW = _NC * _NS
_BPW = _B // _NW
_IW = 128
_NBLK = _V // _IW
_TAILV = _NBLK * _IW
_ROWS_PW = _BPW // _SEQ


def _fmt_body(tblt, tailp, ids, fmt, idxh, ibuf0, ibuf1, obuf0, obuf1,
              idsblk, idxblk, isem0, isem1, osem0, osem1):
    wid = lax.axis_index("s") * _NC + lax.axis_index("c")

    iota16 = lax.iota(jnp.int32, 16)
    _rot = [(iota16 + s) & 15 for s in range(16)]
    _krot = [lax.shift_right_logical(r, 1) for r in _rot]
    _offrot = [lax.shift_left(r & 1, 6) + iota16 for r in _rot]
    _dvecs = [iota16 + 16 * dg for dg in range(4)]

    def transpose_block(ibuf, obuf, nrows):
        ntg = (2 * nrows) // 16

        def tgstep(tg, carry):
            for s in range(16):
                colv = _rot[s] + (16 * tg)
                kv = _krot[s] + (8 * tg)
                for dg in range(4):
                    vals = plsc.load_gather(ibuf, [_dvecs[dg], colv])
                    plsc.store_scatter(obuf, [kv, _offrot[s] + (16 * dg)], vals)
            return carry

        lax.fori_loop(0, ntg, tgstep, 0)

    def sup_body(sup, carry):
        r0 = wid * _ROWS_PW + sup * 8
        pltpu.sync_copy(ids.at[pl.ds(r0, 8)], idsblk)

        def shift16(i, carry2):
            r = i // 64
            c = (i % 64) * 16
            idxblk[r, pl.ds(c, 16)] = lax.shift_right_logical(
                idsblk[r, pl.ds(c, 16)], 1)
            return carry2

        lax.fori_loop(0, 512, shift16, 0)
        pltpu.sync_copy(idxblk, idxh.at[pl.ds(r0, 8)])
        return carry

    lax.fori_loop(0, _ROWS_PW // 8, sup_body, 0)

    def fire_in(c, ibuf, sem):
        pltpu.async_copy(tblt.at[:, pl.ds(c * _IW, _IW)], ibuf, sem)

    def drain_in(ibuf, sem):
        pltpu.make_async_copy(tblt.at[:, pl.ds(0, _IW)], ibuf, sem).wait()

    def fire_out(c, obuf, sem):
        pltpu.async_copy(obuf, fmt.at[pl.ds(c * 64, 64)], sem)

    def drain_out(obuf, sem):
        pltpu.make_async_copy(obuf, fmt.at[pl.ds(0, 64)], sem).wait()

    nblk_w = _NBLK // _NW
    c0 = wid * nblk_w
    fire_in(c0, ibuf0, isem0)

    pltpu.async_copy(obuf0, fmt.at[pl.ds(c0 * 64, 64)], osem0)
    pltpu.async_copy(obuf1, fmt.at[pl.ds(c0 * 64 + 64, 64)], osem1)

    def pairstep(k, carry):
        c = c0 + 2 * k
        drain_in(ibuf0, isem0)
        fire_in(c + 1, ibuf1, isem1)
        drain_out(obuf0, osem0)
        transpose_block(ibuf0, obuf0, 64)
        fire_out(c, obuf0, osem0)
        drain_in(ibuf1, isem1)

        @pl.when(c + 2 < c0 + nblk_w)
        def _():
            fire_in(c + 2, ibuf0, isem0)

        drain_out(obuf1, osem1)
        transpose_block(ibuf1, obuf1, 64)
        fire_out(c + 1, obuf1, osem1)
        return carry

    lax.fori_loop(0, nblk_w // 2, pairstep, 0)
    drain_out(obuf0, osem0)
    drain_out(obuf1, osem1)

    @pl.when(wid < _NBLK - nblk_w * _NW)
    def _():
        c = nblk_w * _NW + wid
        pltpu.sync_copy(tblt.at[:, pl.ds(c * _IW, _IW)], ibuf0)
        transpose_block(ibuf0, obuf0, 64)
        pltpu.sync_copy(obuf0, fmt.at[pl.ds(c * 64, 64)])

    @pl.when(wid == _NW - 1)
    def _():
        pltpu.sync_copy(tailp, ibuf0)
        transpose_block(ibuf0, obuf0, 32)
        pltpu.sync_copy(obuf0.at[pl.ds(0, 32)], fmt.at[pl.ds(_TAILV // 2, 32)])


def _gather_body(ids, idxh, fmt, out3, idsblk, idxblk, pairs0, pairs1,
                 obuf0, obuf1, hbuf, gsem0, gsem1, wsem0, wsem1):
    wid = lax.axis_index("s") * _NC + lax.axis_index("c")
    row0 = wid * _ROWS_PW

    iota16 = lax.iota(jnp.int32, 16)
    _rot = [(iota16 + s) & 15 for s in range(16)]
    _krot = [lax.shift_right_logical(r, 1) for r in _rot]
    _offrot = [lax.shift_left(r & 1, 6) + iota16 for r in _rot]
    _dvecs = [iota16 + 16 * dg for dg in range(4)]

    def fire_g(cc, pairs, sem):
        r = cc // 8
        c = (cc % 8) * _IW
        pltpu.async_copy(fmt.at[idxblk.at[r, pl.ds(c, _IW)]], pairs, sem)

    def drain_g(pairs, sem):
        pltpu.make_async_copy(fmt.at[pl.ds(0, _IW)], pairs, sem).wait()

    def select_t(cc, pairs, obuf, hbuf):
        r = cc // 8
        c = (cc % 8) * _IW
        def hstep(tg, carry):
            hbuf[pl.ds(16 * tg, 16)] = lax.shift_left(
                lax.bitwise_and(idsblk[r, pl.ds(c + 16 * tg, 16)], 1), 6)
            return carry

        lax.fori_loop(0, 8, hstep, 0)

        def tgstep(tg, carry):
            for s in range(16):
                rowv = _rot[s] + (16 * tg)
                hperm = plsc.load_gather(hbuf, [rowv])
                for dg in range(4):
                    colv = hperm + _dvecs[dg]
                    vals = plsc.load_gather(pairs, [rowv, colv])
                    plsc.store_scatter(obuf, [_dvecs[dg], rowv], vals)
            return carry

        lax.fori_loop(0, 8, tgstep, 0)

    def fire_w(cc, sup, obuf, sem):
        row = row0 + sup * 8 + cc // 8
        pltpu.async_copy(obuf, out3.at[row, :, pl.ds((cc % 8) * _IW, _IW)], sem)

    def drain_w(obuf, sem):
        pltpu.make_async_copy(obuf, out3.at[0, :, pl.ds(0, _IW)], sem).wait()

    def sup_body(sup, carry):
        r0 = row0 + sup * 8
        pltpu.sync_copy(ids.at[pl.ds(r0, 8)], idsblk)
        pltpu.sync_copy(idxh.at[pl.ds(r0, 8)], idxblk)

        fire_g(0, pairs0, gsem0)

        def pairstep(k, carry2):
            cc = 2 * k
            drain_g(pairs0, gsem0)
            fire_g(cc + 1, pairs1, gsem1)
            drain_w(obuf0, wsem0)
            select_t(cc, pairs0, obuf0, hbuf)
            fire_w(cc, sup, obuf0, wsem0)
            drain_g(pairs1, gsem1)

            @pl.when(cc + 2 < 64)
            def _():
                fire_g(cc + 2, pairs0, gsem0)

            drain_w(obuf1, wsem1)
            select_t(cc + 1, pairs1, obuf1, hbuf)
            fire_w(cc + 1, sup, obuf1, wsem1)
            return carry2

        lax.fori_loop(0, 32, pairstep, 0)
        return carry

    pltpu.async_copy(obuf0, out3.at[row0, :, pl.ds(0, _IW)], wsem0)
    pltpu.async_copy(obuf1, out3.at[row0, :, pl.ds(_IW, _IW)], wsem1)
    lax.fori_loop(0, _ROWS_PW // 8, sup_body, 0)
    drain_w(obuf0, wsem0)
    drain_w(obuf1, wsem1)


@jax.jit
def kernel(input_ids, attention_mask, embedding_weight):
    tblt = embedding_weight.T
    tail = lax.slice(embedding_weight, (_TAILV, 0), (_V, _D))
    tailp = jnp.concatenate([tail.T, tail.T], axis=1)

    mesh = plsc.VectorSubcoreMesh(core_axis_name="c", subcore_axis_name="s")
    fmt, idxh = pl.kernel(
        _fmt_body,
        mesh=mesh,
        out_type=(
            jax.ShapeDtypeStruct((_V // 2, 128), jnp.float32),
            jax.ShapeDtypeStruct((_BATCH, _SEQ), jnp.int32),
        ),
        scratch_types=[
            pltpu.VMEM((_D, _IW), jnp.float32),
            pltpu.VMEM((_D, _IW), jnp.float32),
            pltpu.VMEM((_D, _IW), jnp.float32),
            pltpu.VMEM((_D, _IW), jnp.float32),
            pltpu.VMEM((8, _SEQ), jnp.int32),
            pltpu.VMEM((8, _SEQ), jnp.int32),
            pltpu.SemaphoreType.DMA,
            pltpu.SemaphoreType.DMA,
            pltpu.SemaphoreType.DMA,
            pltpu.SemaphoreType.DMA,
        ],
        compiler_params=pltpu.CompilerParams(use_tc_tiling_on_sc=True, needs_layout_passes=False),
    )(tblt, tailp, input_ids)

    out3 = pl.kernel(
        _gather_body,
        mesh=mesh,
        out_type=jax.ShapeDtypeStruct((_BATCH, _D, _SEQ), jnp.float32),
        scratch_types=[
            pltpu.VMEM((8, _SEQ), jnp.int32),
            pltpu.VMEM((8, _SEQ), jnp.int32),
            pltpu.VMEM((_IW, 128), jnp.float32),
            pltpu.VMEM((_IW, 128), jnp.float32),
            pltpu.VMEM((_D, _IW), jnp.float32),
            pltpu.VMEM((_D, _IW), jnp.float32),
            pltpu.VMEM((_IW,), jnp.int32),
            pltpu.SemaphoreType.DMA,
            pltpu.SemaphoreType.DMA,
            pltpu.SemaphoreType.DMA,
            pltpu.SemaphoreType.DMA,
        ],
        compiler_params=pltpu.CompilerParams(use_tc_tiling_on_sc=True, needs_layout_passes=False),
    )(input_ids, idxh, fmt)

    token_embeddings = out3.transpose(0, 2, 1)
    return (input_ids, token_embeddings, attention_mask)


# speedup vs baseline: 1.2818x; 1.2740x over previous
import jax
import jax.numpy as jnp
from jax import lax
from jax.experimental import pallas as pl
from jax.experimental.pallas import tpu as pltpu
from jax.experimental.pallas import tpu_sc as plsc

_D = 64
_BATCH = 1024
_SEQ = 1024
_B = _BATCH * _SEQ
_V = 1000000
_NC = 2
_NS = 16
_NW = _NC * _NS
_BPW = _B // _NW
_IW = 128
_NBLK = _V // _IW
_TAILV = _NBLK * _IW
_ROWS_PW = _BPW // _SEQ


def _fmt_body(tblt, tailp, fmt, ibuf0, ibuf1, obuf0, obuf1,
              isem0, isem1, osem0, osem1):
    wid = lax.axis_index("s") * _NC + lax.axis_index("c")

    iota16 = lax.iota(jnp.int32, 16)
    _rot = [(iota16 + s) & 15 for s in range(16)]
    _krot = [lax.shift_right_logical(r, 1) for r in _rot]
    _offrot = [lax.shift_left(r & 1, 6) + iota16 for r in _rot]
    _dvecs = [iota16 + 16 * dg for dg in range(4)]

    def transpose_block(ibuf, obuf, nrows):
        ntg = (2 * nrows) // 16

        def tgstep(tg, carry):
            for s in range(16):
                colv = _rot[s] + (16 * tg)
                kv = _krot[s] + (8 * tg)
                for dg in range(4):
                    vals = plsc.load_gather(ibuf, [_dvecs[dg], colv])
                    plsc.store_scatter(obuf, [kv, _offrot[s] + (16 * dg)], vals)
            return carry

        lax.fori_loop(0, ntg, tgstep, 0)

    def fire_in(c, ibuf, sem):
        pltpu.async_copy(tblt.at[:, pl.ds(c * _IW, _IW)], ibuf, sem)

    def drain_in(ibuf, sem):
        pltpu.make_async_copy(tblt.at[:, pl.ds(0, _IW)], ibuf, sem).wait()

    def fire_out(c, obuf, sem):
        pltpu.async_copy(obuf, fmt.at[pl.ds(c * 64, 64)], sem)

    def drain_out(obuf, sem):
        pltpu.make_async_copy(obuf, fmt.at[pl.ds(0, 64)], sem).wait()

    nblk_w = _NBLK // _NW
    c0 = wid * nblk_w
    fire_in(c0, ibuf0, isem0)

    pltpu.async_copy(obuf0, fmt.at[pl.ds(c0 * 64, 64)], osem0)
    pltpu.async_copy(obuf1, fmt.at[pl.ds(c0 * 64 + 64, 64)], osem1)

    def pairstep(k, carry):
        c = c0 + 2 * k
        drain_in(ibuf0, isem0)
        fire_in(c + 1, ibuf1, isem1)
        drain_out(obuf0, osem0)
        transpose_block(ibuf0, obuf0, 64)
        fire_out(c, obuf0, osem0)
        drain_in(ibuf1, isem1)

        @pl.when(c + 2 < c0 + nblk_w)
        def _():
            fire_in(c + 2, ibuf0, isem0)

        drain_out(obuf1, osem1)
        transpose_block(ibuf1, obuf1, 64)
        fire_out(c + 1, obuf1, osem1)
        return carry

    lax.fori_loop(0, nblk_w // 2, pairstep, 0)
    drain_out(obuf0, osem0)
    drain_out(obuf1, osem1)

    @pl.when(wid < _NBLK - nblk_w * _NW)
    def _():
        c = nblk_w * _NW + wid
        pltpu.sync_copy(tblt.at[:, pl.ds(c * _IW, _IW)], ibuf0)
        transpose_block(ibuf0, obuf0, 64)
        pltpu.sync_copy(obuf0, fmt.at[pl.ds(c * 64, 64)])

    @pl.when(wid == _NW - 1)
    def _():
        pltpu.sync_copy(tailp, ibuf0)
        transpose_block(ibuf0, obuf0, 32)
        pltpu.sync_copy(obuf0.at[pl.ds(0, 32)], fmt.at[pl.ds(_TAILV // 2, 32)])


def _gather_body(ids, tbl_lin, out5, idsblk, rows0, rows1, obuf0, obuf1,
                 gsem0, gsem1, wsem0, wsem1):
    wid = lax.axis_index("s") * _NC + lax.axis_index("c")
    row0 = wid * _ROWS_PW

    iota16 = lax.iota(jnp.int32, 16)
    _rot = [(iota16 + s) & 15 for s in range(16)]
    _dvecs = [iota16 + 16 * dg for dg in range(4)]
    _ivecs = [lax.shift_right_logical(d, 3) for d in _dvecs]
    _rvecs = [d & 7 for d in _dvecs]

    def fire_g(cc, rows, sem):
        r = cc // 8
        c = (cc % 8) * _IW
        pltpu.async_copy(tbl_lin.at[idsblk.at[r, pl.ds(c, _IW)]], rows, sem)

    def drain_g(rows, sem):
        pltpu.make_async_copy(tbl_lin.at[pl.ds(0, _IW)], rows, sem).wait()

    def transp(rows, obuf):
        def tgstep(tg, carry):
            for s in range(16):
                tv = _rot[s] + (16 * tg)
                for dg in range(4):
                    vals = plsc.load_gather(rows, [tv, _dvecs[dg]])
                    plsc.store_scatter(obuf, [_ivecs[dg], _rvecs[dg], tv], vals)
            return carry

        lax.fori_loop(0, 8, tgstep, 0)

    def fire_w(cc, sup, obuf, sem):
        b = row0 + sup * 8 + cc // 8
        pltpu.async_copy(obuf, out5.at[b, :, cc % 8], sem)

    def drain_w(obuf, sem):
        pltpu.make_async_copy(obuf, out5.at[0, :, 0], sem).wait()

    def sup_body(sup, carry):
        pltpu.sync_copy(ids.at[pl.ds(row0 + sup * 8, 8)], idsblk)
        fire_g(0, rows0, gsem0)

        def pairstep(k, carry2):
            cc = 2 * k
            drain_g(rows0, gsem0)
            fire_g(cc + 1, rows1, gsem1)
            drain_w(obuf0, wsem0)
            transp(rows0, obuf0)
            fire_w(cc, sup, obuf0, wsem0)
            drain_g(rows1, gsem1)

            @pl.when(cc + 2 < 64)
            def _():
                fire_g(cc + 2, rows0, gsem0)

            drain_w(obuf1, wsem1)
            transp(rows1, obuf1)
            fire_w(cc + 1, sup, obuf1, wsem1)
            return carry2

        lax.fori_loop(0, 32, pairstep, 0)
        return carry

    pltpu.async_copy(obuf0, out5.at[row0, :, 0], wsem0)
    pltpu.async_copy(obuf1, out5.at[row0, :, 1], wsem1)
    lax.fori_loop(0, _ROWS_PW // 8, sup_body, 0)
    drain_w(obuf0, wsem0)
    drain_w(obuf1, wsem1)


@jax.jit
def kernel(input_ids, attention_mask, embedding_weight):
    tblt = embedding_weight.T
    tail = lax.slice(embedding_weight, (_TAILV, 0), (_V, _D))
    tailp = jnp.concatenate([tail.T, tail.T], axis=1)

    mesh = plsc.VectorSubcoreMesh(core_axis_name="c", subcore_axis_name="s")
    fmt = pl.kernel(
        _fmt_body,
        mesh=mesh,
        out_type=jax.ShapeDtypeStruct((_V // 2, 128), jnp.float32),
        scratch_types=[
            pltpu.VMEM((_D, _IW), jnp.float32),
            pltpu.VMEM((_D, _IW), jnp.float32),
            pltpu.VMEM((_D, _IW), jnp.float32),
            pltpu.VMEM((_D, _IW), jnp.float32),
            pltpu.SemaphoreType.DMA,
            pltpu.SemaphoreType.DMA,
            pltpu.SemaphoreType.DMA,
            pltpu.SemaphoreType.DMA,
        ],
        compiler_params=pltpu.CompilerParams(
            use_tc_tiling_on_sc=True, needs_layout_passes=False),
    )(tblt, tailp)

    tbl_lin = fmt.reshape(_V, _D)

    out5 = pl.kernel(
        _gather_body,
        mesh=mesh,
        out_type=jax.ShapeDtypeStruct((_BATCH, 8, 8, 8, 128), jnp.float32),
        scratch_types=[
            pltpu.VMEM((8, _SEQ), jnp.int32),
            pltpu.VMEM((_IW, _D), jnp.float32),
            pltpu.VMEM((_IW, _D), jnp.float32),
            pltpu.VMEM((8, 8, 128), jnp.float32),
            pltpu.VMEM((8, 8, 128), jnp.float32),
            pltpu.SemaphoreType.DMA,
            pltpu.SemaphoreType.DMA,
            pltpu.SemaphoreType.DMA,
            pltpu.SemaphoreType.DMA,
        ],
        compiler_params=pltpu.CompilerParams(
            use_tc_tiling_on_sc=False, needs_layout_passes=False),
    )(input_ids, tbl_lin)

    token_embeddings = out5.transpose(0, 2, 4, 1, 3).reshape(_BATCH, _SEQ, _D)
    return (input_ids, token_embeddings, attention_mask)
